# baseline (speedup 1.0000x reference)
import functools

import jax
import jax.numpy as jnp
from jax import lax
from jax.experimental import pallas as pl
from jax.experimental.pallas import tpu as pltpu
from jax.experimental.pallas import tpu_sc as plsc

B = 16
S = 512
D = 128
N = 4096
NE = 63
Q = B * S
QT = 1024
NSTEPS = Q // QT
LQ = QT // 8
NCH = N // 128
OFF = 23814.0
NW = 32
BPW = Q // NW


def _tc_body(x_ref, e_ref, nk0_ref, nk1_ref, nk2_ref, out_ref,
             dc0_s, dc1_s, dc2_s, m0_s, m1_s, m2_s, b_s):
    t = pl.program_id(0)

    @pl.when(t == 0)
    def _prep():
        xv = x_ref[...]
        acc = jnp.zeros((Q // 128, 128), jnp.float32)
        for j in range(NE):
            acc += (xv >= e_ref[0, j]).astype(jnp.float32)
        dc0_s[...] = acc
        nxt = pltpu.roll(acc, Q // 128 - 1, 0)
        dc1_s[...] = jnp.concatenate([acc[:, 1:], nxt[:, :1]], axis=1)
        dc2_s[...] = jnp.concatenate([acc[:, 2:], nxt[:, :2]], axis=1)

        nk0 = nk0_ref[...]
        nk1 = nk1_ref[...]
        nk2 = nk2_ref[...]
        n2 = nk0 * nk0 + nk1 * nk1 + nk2 * nk2
        chid = lax.broadcasted_iota(
            jnp.int32, (NCH, 128), 0).astype(jnp.float32)
        bv = (n2 + OFF) * 32.0 + chid
        m0 = nk0 * -64.0
        m1 = nk1 * -64.0
        m2 = nk2 * -64.0
        for ch in range(NCH):
            sl = pl.ds(ch * 8, 8)
            m0_s[sl, :] = jnp.broadcast_to(m0[ch:ch + 1, :], (8, 128))
            m1_s[sl, :] = jnp.broadcast_to(m1[ch:ch + 1, :], (8, 128))
            m2_s[sl, :] = jnp.broadcast_to(m2[ch:ch + 1, :], (8, 128))
            b_s[sl, :] = jnp.broadcast_to(bv[ch:ch + 1, :], (8, 128))

    rows = pl.ds(t * 8, 8)
    dcv0 = dc0_s[rows, :]
    dcv1 = dc1_s[rows, :]
    dcv2 = dc2_s[rows, :]
    lane = lax.broadcasted_iota(jnp.int32, (8, 128), 1).astype(jnp.float32)
    d0 = []
    d1 = []
    d2 = []
    cmin = [None] * LQ
    for q in range(LQ):
        d0.append(jnp.broadcast_to(dcv0[:, q:q + 1], (8, 128)))
        d1.append(jnp.broadcast_to(dcv1[:, q:q + 1], (8, 128)))
        d2.append(jnp.broadcast_to(dcv2[:, q:q + 1], (8, 128)))
    for ch in range(NCH):
        sl = pl.ds(ch * 8, 8)
        m0 = m0_s[sl, :]
        m1 = m1_s[sl, :]
        m2 = m2_s[sl, :]
        bv = b_s[sl, :]
        for q in range(LQ):
            c = bv + m0 * d0[q] + m1 * d1[q] + m2 * d2[q]
            cmin[q] = c if cmin[q] is None else jnp.minimum(cmin[q], c)
    for q in range(LQ):
        cm = cmin[q]
        mv = jnp.min(cm, axis=1, keepdims=True)
        lv = jnp.min(jnp.where(cm == mv, lane, 128.0),
                     axis=1, keepdims=True)
        nid = (mv.astype(jnp.int32) & 31) * 128 + lv.astype(jnp.int32)
        out_ref[0, :, q:q + 1] = nid


HPW = BPW // 2


def _sc_gather_body(idx_hbm, tab_hbm, out_hbm,
                    idx_v0, idx_v1, rows_v0, rows_v1,
                    gsem0, gsem1, ssem0, ssem1):
    c = lax.axis_index("c")
    s = lax.axis_index("s")
    wid = s * 2 + c
    base = wid * BPW
    t = wid // 4
    s0 = (wid % 4) * 2
    pltpu.sync_copy(idx_hbm.at[t, s0], idx_v0)
    g0 = pltpu.async_copy(tab_hbm.at[idx_v0], rows_v0, gsem0)
    pltpu.sync_copy(idx_hbm.at[t, s0 + 1], idx_v1)
    g1 = pltpu.async_copy(tab_hbm.at[idx_v1], rows_v1, gsem1)
    g0.wait()
    s0 = pltpu.async_copy(rows_v0, out_hbm.at[pl.ds(base, HPW)], ssem0)
    g1.wait()

    @pl.when(wid % 2 == 1)
    def _zero_pad_rows():
        z = jnp.zeros((16,), jnp.float32)
        for r in (HPW - 2, HPW - 1):
            for cc in range(D // 16):
                rows_v1[r, pl.ds(cc * 16, 16)] = z

    s1 = pltpu.async_copy(rows_v1, out_hbm.at[pl.ds(base + HPW, HPW)], ssem1)
    s0.wait()
    s1.wait()


def _make_sc_gather():
    return functools.partial(
        pl.kernel,
        out_type=jax.ShapeDtypeStruct((Q, D), jnp.float32),
        mesh=plsc.VectorSubcoreMesh(core_axis_name="c", subcore_axis_name="s"),
        scratch_types=[
            pltpu.VMEM((HPW,), jnp.int32),
            pltpu.VMEM((HPW,), jnp.int32),
            pltpu.VMEM((HPW, D), jnp.float32),
            pltpu.VMEM((HPW, D), jnp.float32),
            pltpu.SemaphoreType.DMA,
            pltpu.SemaphoreType.DMA,
            pltpu.SemaphoreType.DMA,
            pltpu.SemaphoreType.DMA,
        ],
    )(_sc_gather_body)


def kernel(x, bin_edges, node_keys, graph_emb):
    x2d = x.reshape(Q // 128, 128)
    ev = bin_edges.reshape(1, NE)

    sc64 = pltpu.VMEM((Q // 128, 128), jnp.float32)
    screp = pltpu.VMEM((8 * NCH, 128), jnp.float32)
    nid = pl.pallas_call(
        _tc_body,
        grid=(NSTEPS,),
        in_specs=[
            pl.BlockSpec((Q // 128, 128), lambda t: (0, 0)),
            pl.BlockSpec(memory_space=pltpu.SMEM),
            pl.BlockSpec((NCH, 128), lambda t: (0, 0)),
            pl.BlockSpec((NCH, 128), lambda t: (0, 0)),
            pl.BlockSpec((NCH, 128), lambda t: (0, 0)),
        ],
        out_specs=pl.BlockSpec((1, 8, LQ), lambda t: (t, 0, 0)),
        out_shape=jax.ShapeDtypeStruct((NSTEPS, 8, LQ), jnp.int32),
        scratch_shapes=[sc64, sc64, sc64, screp, screp, screp, screp],
    )(x2d, ev,
      node_keys[:, 0].reshape(NCH, 128),
      node_keys[:, 1].reshape(NCH, 128),
      node_keys[:, 2].reshape(NCH, 128))

    out = _make_sc_gather()(nid, graph_emb)
    return out.reshape(B, S, D)


# speedup vs baseline: 1.0004x; 1.0004x over previous
import functools

import jax
import jax.numpy as jnp
from jax import lax
from jax.experimental import pallas as pl
from jax.experimental.pallas import tpu as pltpu
from jax.experimental.pallas import tpu_sc as plsc

B = 16
S = 512
D = 128
N = 4096
NE = 63
Q = B * S
QT = 1024
NSTEPS = Q // QT
LQ = QT // 8
NCH = N // 128
OFF = 23814.0
NW = 32
BPW = Q // NW


def _tc_body(x_ref, e_ref, nk0_ref, nk1_ref, nk2_ref, out_ref,
             dc0_s, dc1_s, dc2_s, m0_s, m1_s, m2_s, b_s):
    t = pl.program_id(0)

    @pl.when(t == 0)
    def _prep():
        xv = x_ref[...]
        acc = jnp.zeros((Q // 128, 128), jnp.float32)
        for j in range(NE):
            acc += (xv >= e_ref[0, j]).astype(jnp.float32)
        dc0_s[...] = acc
        nxt = pltpu.roll(acc, Q // 128 - 1, 0)
        dc1_s[...] = jnp.concatenate([acc[:, 1:], nxt[:, :1]], axis=1)
        dc2_s[...] = jnp.concatenate([acc[:, 2:], nxt[:, :2]], axis=1)

        nk0 = nk0_ref[...]
        nk1 = nk1_ref[...]
        nk2 = nk2_ref[...]
        n2 = nk0 * nk0 + nk1 * nk1 + nk2 * nk2
        chid = lax.broadcasted_iota(
            jnp.int32, (NCH, 128), 0).astype(jnp.float32)
        bv = (n2 + OFF) * 32.0 + chid
        m0 = nk0 * -64.0
        m1 = nk1 * -64.0
        m2 = nk2 * -64.0
        for ch in range(NCH):
            sl = pl.ds(ch * 8, 8)
            m0_s[sl, :] = jnp.broadcast_to(m0[ch:ch + 1, :], (8, 128))
            m1_s[sl, :] = jnp.broadcast_to(m1[ch:ch + 1, :], (8, 128))
            m2_s[sl, :] = jnp.broadcast_to(m2[ch:ch + 1, :], (8, 128))
            b_s[sl, :] = jnp.broadcast_to(bv[ch:ch + 1, :], (8, 128))

    rows = pl.ds(t * 8, 8)
    dcv0 = dc0_s[rows, :]
    dcv1 = dc1_s[rows, :]
    dcv2 = dc2_s[rows, :]
    lane = lax.broadcasted_iota(jnp.int32, (8, 128), 1).astype(jnp.float32)
    d0 = []
    d1 = []
    d2 = []
    cmin = [None] * LQ
    for q in range(LQ):
        d0.append(jnp.broadcast_to(dcv0[:, q:q + 1], (8, 128)))
        d1.append(jnp.broadcast_to(dcv1[:, q:q + 1], (8, 128)))
        d2.append(jnp.broadcast_to(dcv2[:, q:q + 1], (8, 128)))
    for ch in range(NCH):
        sl = pl.ds(ch * 8, 8)
        m0 = m0_s[sl, :]
        m1 = m1_s[sl, :]
        m2 = m2_s[sl, :]
        bv = b_s[sl, :]
        for q in range(LQ):
            c = bv + m0 * d0[q] + m1 * d1[q] + m2 * d2[q]
            cmin[q] = c if cmin[q] is None else jnp.minimum(cmin[q], c)
    for q in range(LQ):
        cm = cmin[q]
        mv = jnp.min(cm, axis=1, keepdims=True)
        lv = jnp.min(jnp.where(cm == mv, lane, 128.0),
                     axis=1, keepdims=True)
        nid = (mv.astype(jnp.int32) & 31) * 128 + lv.astype(jnp.int32)
        out_ref[0, :, q:q + 1] = nid


HPW = BPW // 2


def _sc_gather_body(idx_hbm, tab_hbm, out_hbm,
                    idx_v0, idx_v1, rows_v0, rows_v1,
                    gsem0, gsem1, ssem0, ssem1):
    c = lax.axis_index("c")
    s = lax.axis_index("s")
    wid = s * 2 + c
    base = wid * BPW
    t = wid // 4
    row0 = (wid % 4) * 2
    pltpu.sync_copy(idx_hbm.at[t, row0], idx_v0)
    g0 = pltpu.async_copy(tab_hbm.at[idx_v0], rows_v0, gsem0)
    pltpu.sync_copy(idx_hbm.at[t, row0 + 1], idx_v1)
    g1 = pltpu.async_copy(tab_hbm.at[idx_v1], rows_v1, gsem1)
    g0.wait()
    st0 = pltpu.async_copy(rows_v0, out_hbm.at[pl.ds(base, HPW)], ssem0)
    g1.wait()

    @pl.when(wid % 2 == 1)
    def _zero_pad_rows():
        z = jnp.zeros((16,), jnp.float32)
        for r in (HPW - 2, HPW - 1):
            for cc in range(D // 16):
                rows_v1[r, pl.ds(cc * 16, 16)] = z

    st1 = pltpu.async_copy(rows_v1, out_hbm.at[pl.ds(base + HPW, HPW)], ssem1)
    st0.wait()
    st1.wait()


def _make_sc_gather():
    return functools.partial(
        pl.kernel,
        out_type=jax.ShapeDtypeStruct((Q, D), jnp.float32),
        mesh=plsc.VectorSubcoreMesh(core_axis_name="c", subcore_axis_name="s"),
        scratch_types=[
            pltpu.VMEM((HPW,), jnp.int32),
            pltpu.VMEM((HPW,), jnp.int32),
            pltpu.VMEM((HPW, D), jnp.float32),
            pltpu.VMEM((HPW, D), jnp.float32),
            pltpu.SemaphoreType.DMA,
            pltpu.SemaphoreType.DMA,
            pltpu.SemaphoreType.DMA,
            pltpu.SemaphoreType.DMA,
        ],
    )(_sc_gather_body)


def kernel(x, bin_edges, node_keys, graph_emb):
    x2d = x.reshape(Q // 128, 128)
    ev = bin_edges.reshape(1, NE)

    sc64 = pltpu.VMEM((Q // 128, 128), jnp.float32)
    screp = pltpu.VMEM((8 * NCH, 128), jnp.float32)
    nid = pl.pallas_call(
        _tc_body,
        grid=(NSTEPS,),
        in_specs=[
            pl.BlockSpec((Q // 128, 128), lambda t: (0, 0)),
            pl.BlockSpec(memory_space=pltpu.SMEM),
            pl.BlockSpec((NCH, 128), lambda t: (0, 0)),
            pl.BlockSpec((NCH, 128), lambda t: (0, 0)),
            pl.BlockSpec((NCH, 128), lambda t: (0, 0)),
        ],
        out_specs=pl.BlockSpec((1, 8, LQ), lambda t: (t, 0, 0)),
        out_shape=jax.ShapeDtypeStruct((NSTEPS, 8, LQ), jnp.int32),
        scratch_shapes=[sc64, sc64, sc64, screp, screp, screp, screp],
    )(x2d, ev,
      node_keys[:, 0].reshape(NCH, 128),
      node_keys[:, 1].reshape(NCH, 128),
      node_keys[:, 2].reshape(NCH, 128))

    out = _make_sc_gather()(nid, graph_emb)
    return out.reshape(B, S, D)
